# Initial kernel scaffold; baseline (speedup 1.0000x reference)
import jax
import jax.numpy as jnp
from jax.experimental import pallas as pl


def kernel(x, params):
    raise NotImplementedError("write your pallas kernel here")


# speedup vs baseline: 1.2380x; 1.2380x over previous
import functools

import jax
import jax.numpy as jnp
from jax import lax
from jax.experimental import pallas as pl
from jax.experimental.pallas import tpu as pltpu

_BINS = 256
_INV_S2 = 1.0e4
_BN_EPS = 1e-5


def _hist_body(x_ref, o_ref):
    centers = lax.broadcasted_iota(jnp.int32, (_BINS, 128), 0).astype(
        jnp.float32
    ) * (1.0 / 255.0)

    def body(i, acc):
        xt = x_ref[0, pl.ds(i * 8, 8), :]
        for s in range(8):
            d = xt[s : s + 1, :] - centers
            acc = acc + jnp.exp(d * d * (-_INV_S2))
        return acc

    acc = lax.fori_loop(0, 49, body, jnp.zeros((_BINS, 128), jnp.float32))

    ident = jnp.where(
        lax.broadcasted_iota(jnp.int32, (_BINS, _BINS), 0)
        == lax.broadcasted_iota(jnp.int32, (_BINS, _BINS), 1),
        1.0,
        0.0,
    )
    acc_t = lax.dot_general(
        acc, ident, (((0,), (0,)), ((), ())), preferred_element_type=jnp.float32
    )
    hist = jnp.sum(acc_t, axis=0, keepdims=True)
    total = jnp.sum(hist, axis=1, keepdims=True)
    pdf = hist / (total + 1e-6)
    upper = jnp.where(
        lax.broadcasted_iota(jnp.int32, (_BINS, _BINS), 0)
        <= lax.broadcasted_iota(jnp.int32, (_BINS, _BINS), 1),
        1.0,
        0.0,
    )
    cdf = jnp.dot(pdf, upper, preferred_element_type=jnp.float32)
    o_ref[0] = cdf


def _soft_cdf(x):
    B, C, H, W = x.shape
    xr = x.reshape(B * C, 392, 128)
    out = pl.pallas_call(
        _hist_body,
        grid=(B * C,),
        in_specs=[pl.BlockSpec((1, 392, 128), lambda i: (i, 0, 0))],
        out_specs=pl.BlockSpec((1, 1, _BINS), lambda i: (i, 0, 0)),
        out_shape=jax.ShapeDtypeStruct((B * C, 1, _BINS), jnp.float32),
        compiler_params=pltpu.CompilerParams(
            dimension_semantics=("parallel",),
        ),
    )(xr)
    return out.reshape(B, C * _BINS)


def _c(x, w, stride, pad):
    return lax.conv_general_dilated(
        x, w, (stride, stride), [(pad, pad), (pad, pad)],
        dimension_numbers=("NCHW", "OIHW", "NCHW"))


def _bnorm(x, p):
    g = p["g"][None, :, None, None]
    b = p["b"][None, :, None, None]
    m = p["m"][None, :, None, None]
    v = p["v"][None, :, None, None]
    return (x - m) * g * lax.rsqrt(v + _BN_EPS) + b


def _blk(x, p, stride):
    out = jax.nn.relu(_bnorm(_c(x, p["conv1"], stride, 1), p["bn1"]))
    out = _bnorm(_c(out, p["conv2"], 1, 1), p["bn2"])
    if "down" in p:
        sc = _bnorm(_c(x, p["down"], stride, 0), p["dbn"])
    else:
        sc = x
    return jax.nn.relu(out + sc)


def kernel(x, params):
    B = x.shape[0]
    cdf = _soft_cdf(x)

    h = jax.nn.relu(_bnorm(_c(x, params["conv1"], 2, 3), params["bn1"]))
    h = lax.reduce_window(h, -jnp.inf, lax.max,
                          (1, 1, 3, 3), (1, 1, 2, 2),
                          [(0, 0), (0, 0), (1, 1), (1, 1)])
    strides = [1, 2, 2, 2]
    for si, blocks in enumerate(params["layers"]):
        for bi, blk in enumerate(blocks):
            h = _blk(h, blk, strides[si] if bi == 0 else 1)
    spatial = h.mean(axis=(2, 3))

    return jnp.concatenate([cdf, spatial], axis=1)
